# baseline (speedup 1.0000x reference)
import jax
import jax.numpy as jnp
from jax import lax
from jax.experimental import pallas as pl
from jax.experimental.pallas import tpu as pltpu
from jax.experimental.pallas import tpu_sc as plsc

_N = 100000
_D = 128
_H = 5
_B = 100000
_R = _H + 2

_NC, _NS = 2, 16
_NW = _NC * _NS
_S = 2
_BPAD = 102400
_BS = _BPAD // _S
_W = _BS // _NW
_C = 32
_NCH = _W // _C
_CR = _C * _R
assert _NCH % 2 == 0 and _C % 16 == 0

_BLK = 3200
_TG = _BS // _BLK


def _sc_body(emb, dtab, idx_all, res_out, d_out,
             ib0, ib1, rows0, rows1, rv0, rv1, rv2, rv3, rv4, rv5, del_v,
             isem0, isem1, gsem0, gsem1):
    res_v = (rv0, rv1, rv2, rv3, rv4, rv5)
    wid = lax.axis_index("s") * _NC + lax.axis_index("c")
    ibase = pl.multiple_of(wid * _NCH * _CR, 8)
    bufs = ((ib0, rows0, isem0, gsem0), (ib1, rows1, isem1, gsem1))
    lane = lax.iota(jnp.int32, 16)

    def idx_slice(g):
        return idx_all.at[pl.ds(ibase + g * _CR, _CR)]

    def fire_gathers(g, bs):
        ib, rows, _, gsem = bs
        for k in range(_R):
            pltpu.async_copy(emb.at[ib.at[pl.ds(k * _C, _C)]],
                             rows.at[pl.ds(k * _C, _C)], gsem)
        pltpu.async_copy(dtab.at[ib.at[pl.ds(0, _C)]],
                         del_v.at[pl.ds(g * _C, _C)], gsem)

    def drain_gathers(bs):
        ib, rows, _, gsem = bs
        for k in range(_R):
            pltpu.make_async_copy(emb.at[ib.at[pl.ds(0, _C)]],
                                  rows.at[pl.ds(k * _C, _C)], gsem).wait()
        pltpu.make_async_copy(dtab.at[ib.at[pl.ds(0, _C)]],
                              del_v.at[pl.ds(0, _C)], gsem).wait()

    def compute(g, bs):
        _, rows, _, _ = bs
        for grp in range(_C // 16):
            e0 = grp * 16
            rs = lane + e0

            def dstep(d, accs):
                cidx = jnp.full((16,), d, dtype=jnp.int32)
                sv = plsc.load_gather(rows, [rs, cidx])
                tv = plsc.load_gather(rows, [rs + _C, cidx])
                dst = sv - tv
                new = [accs[0] + dst * dst]
                for j in range(_H):
                    hv = plsc.load_gather(rows, [rs + (2 + j) * _C, cidx])
                    dh = sv - hv
                    new.append(accs[1 + j] + dh * dh)
                return tuple(new)

            zero = jnp.zeros((16,), jnp.float32)
            accs = lax.fori_loop(0, _D, dstep, (zero,) * 6)
            off = g * _C + e0
            for k in range(6):
                res_v[k][pl.ds(off, 16)] = accs[k]

    pltpu.async_copy(idx_slice(0), ib0, isem0)
    pltpu.async_copy(idx_slice(1), ib1, isem1)
    pltpu.make_async_copy(idx_slice(0), ib0, isem0).wait()
    fire_gathers(0, bufs[0])

    def outer(i, carry):
        for b in range(2):
            g = 2 * i + b
            nb = 1 - b
            drain_gathers(bufs[b])

            @pl.when(g + 2 < _NCH)
            def _fi():
                pltpu.async_copy(idx_slice(g + 2), bufs[b][0], bufs[b][2])

            @pl.when(g + 1 < _NCH)
            def _fg():
                pltpu.make_async_copy(idx_slice(0), bufs[nb][0],
                                      bufs[nb][2]).wait()
                fire_gathers(g + 1, bufs[nb])

            compute(g, bufs[b])
        return carry

    lax.fori_loop(0, _NCH // 2, outer, 0)
    base = pl.multiple_of(wid * _W, 8)
    for k in range(6):
        pltpu.sync_copy(res_v[k], res_out.at[pl.ds(k * _BS + base, _W)])
    pltpu.sync_copy(del_v, d_out.at[pl.ds(base, _W)])


def _tail_body(res_ref, delta_ref, edge_ref, hst_ref, mask_ref,
               sign_ref, loss_ref):
    r = res_ref[...]
    p_mu = -r[0:1]
    alpha = -r[1:1 + _H]
    m = jnp.max(alpha, axis=0, keepdims=True)
    es = jnp.exp(alpha - m)
    z_norm = jnp.sum(es, axis=0, keepdims=True)
    d_time = edge_ref[...] - hst_ref[...]
    wgt = (es / z_norm) * alpha * jnp.exp(-delta_ref[...] * d_time) * mask_ref[...]
    p_lambda = p_mu + jnp.sum(wgt, axis=0, keepdims=True)
    z = sign_ref[0] * p_lambda
    loss_ref[...] = jnp.maximum(-z, 0.0) + jnp.log(1.0 + jnp.exp(-jnp.abs(z)))


_sc_mesh = plsc.VectorSubcoreMesh(core_axis_name="c", subcore_axis_name="s")

_sc_dist = pl.kernel(
    _sc_body,
    out_type=(
        jax.ShapeDtypeStruct((6 * _BS,), jnp.float32),
        jax.ShapeDtypeStruct((_BS,), jnp.float32),
    ),
    mesh=_sc_mesh,
    compiler_params=pltpu.CompilerParams(needs_layout_passes=False),
    scratch_types=[
        pltpu.VMEM((_CR,), jnp.int32),
        pltpu.VMEM((_CR,), jnp.int32),
        pltpu.VMEM((_CR, _D), jnp.float32),
        pltpu.VMEM((_CR, _D), jnp.float32),
        pltpu.VMEM((_W,), jnp.float32),
        pltpu.VMEM((_W,), jnp.float32),
        pltpu.VMEM((_W,), jnp.float32),
        pltpu.VMEM((_W,), jnp.float32),
        pltpu.VMEM((_W,), jnp.float32),
        pltpu.VMEM((_W,), jnp.float32),
        pltpu.VMEM((_W,), jnp.float32),
        pltpu.SemaphoreType.DMA,
        pltpu.SemaphoreType.DMA,
        pltpu.SemaphoreType.DMA,
        pltpu.SemaphoreType.DMA,
    ],
)

_tail = pl.pallas_call(
    _tail_body,
    grid=(_TG,),
    in_specs=[
        pl.BlockSpec((6, _BLK), lambda i: (0, i)),
        pl.BlockSpec((1, _BLK), lambda i: (0, i)),
        pl.BlockSpec((1, _BLK), lambda i: (0, i)),
        pl.BlockSpec((_H, _BLK), lambda i: (0, i)),
        pl.BlockSpec((_H, _BLK), lambda i: (0, i)),
        pl.BlockSpec(memory_space=pltpu.SMEM),
    ],
    out_specs=pl.BlockSpec((1, _BLK), lambda i: (0, i)),
    out_shape=jax.ShapeDtypeStruct((1, _BS), jnp.float32),
)


def _make_idx(s32, t32, h32):
    s_r = s32.reshape(_NW * _NCH, 1, _C)
    t_r = t32.reshape(_NW * _NCH, 1, _C)
    h_r = h32.reshape(_NW * _NCH, _C, _H).transpose(0, 2, 1)
    return jnp.concatenate([s_r, t_r, h_r], axis=1).reshape(-1)


def kernel(sign, s, t, edge_times_batch, h_s, h_s_times, h_s_mask, emb_table,
           delta_table):
    def pad(x):
        return jnp.concatenate(
            [x, jnp.zeros((_BPAD - _B,) + x.shape[1:], x.dtype)], axis=0)

    def pad_idx(x):
        npad = _BPAD - x.shape[0]
        extra = (jnp.arange(npad * (x.size // x.shape[0]), dtype=jnp.int32)
                 % _N).reshape((npad,) + x.shape[1:])
        return jnp.concatenate([x.astype(jnp.int32), extra], axis=0)

    sp, tp, hp = pad_idx(s), pad_idx(t), pad_idx(h_s)
    hstp, maskp = pad(h_s_times), pad(h_s_mask)
    edgep = pad(edge_times_batch)
    dflat = delta_table.reshape(_N)

    losses = []
    for k in range(_S):
        lo = k * _BS
        idx_all = _make_idx(sp[lo:lo + _BS], tp[lo:lo + _BS], hp[lo:lo + _BS])
        res, delta_g = _sc_dist(emb_table, dflat, idx_all)
        losses.append(_tail(res.reshape(6, _BS), delta_g.reshape(1, _BS),
                            edgep[lo:lo + _BS].reshape(1, _BS),
                            hstp[lo:lo + _BS].T, maskp[lo:lo + _BS].T, sign))
    loss = jnp.concatenate(losses, axis=1)
    return loss[0, :_B]


# speedup vs baseline: 1.0907x; 1.0907x over previous
import jax
import jax.numpy as jnp
from jax import lax
from jax.experimental import pallas as pl
from jax.experimental.pallas import tpu as pltpu
from jax.experimental.pallas import tpu_sc as plsc

_N = 100000
_D = 128
_H = 5
_B = 100000
_R = _H + 2

_NC, _NS = 2, 16
_NW = _NC * _NS
_S = 2
_BPAD = 102400
_BS = _BPAD // _S
_W = _BS // _NW
_C = 32
_NCH = _W // _C
_CR = _C * _R
assert _NCH % 2 == 0 and _C % 16 == 0

_BLK = 3200
_TG = _BS // _BLK


def _sc_body(emb, dtab, idx_all, res_out, d_out,
             ib0, ib1, rows0, rows1, rv0, rv1, rv2, rv3, rv4, rv5, del_v,
             isem0, isem1, gsem0, gsem1):
    res_v = (rv0, rv1, rv2, rv3, rv4, rv5)
    wid = lax.axis_index("s") * _NC + lax.axis_index("c")
    ibase = pl.multiple_of(wid * _NCH * _CR, 8)
    bufs = ((ib0, rows0, isem0, gsem0), (ib1, rows1, isem1, gsem1))
    lane = lax.iota(jnp.int32, 16)

    def idx_slice(g):
        return idx_all.at[pl.ds(ibase + g * _CR, _CR)]

    def fire_gathers(g, bs):
        ib, rows, _, gsem = bs
        for k in range(_R):
            pltpu.async_copy(emb.at[ib.at[pl.ds(k * _C, _C)]],
                             rows.at[pl.ds(k * _C, _C)], gsem)
        pltpu.async_copy(dtab.at[ib.at[pl.ds(0, _C)]],
                         del_v.at[pl.ds(g * _C, _C)], gsem)

    def drain_gathers(bs):
        ib, rows, _, gsem = bs
        for k in range(_R):
            pltpu.make_async_copy(emb.at[ib.at[pl.ds(0, _C)]],
                                  rows.at[pl.ds(k * _C, _C)], gsem).wait()
        pltpu.make_async_copy(dtab.at[ib.at[pl.ds(0, _C)]],
                              del_v.at[pl.ds(0, _C)], gsem).wait()

    def compute(g, bs):
        _, rows, _, _ = bs
        for grp in range(_C // 16):
            e0 = grp * 16
            rs = lane + e0

            zero = jnp.zeros((16,), jnp.float32)

            @plsc.parallel_loop(0, _D, unroll=8, carry=(zero,) * 6)
            def accs(d, accs_in):
                cidx = jnp.full((16,), d, dtype=jnp.int32)
                sv = plsc.load_gather(rows, [rs, cidx])
                tv = plsc.load_gather(rows, [rs + _C, cidx])
                dst = sv - tv
                new = [accs_in[0] + dst * dst]
                for j in range(_H):
                    hv = plsc.load_gather(rows, [rs + (2 + j) * _C, cidx])
                    dh = sv - hv
                    new.append(accs_in[1 + j] + dh * dh)
                return tuple(new)
            off = g * _C + e0
            for k in range(6):
                res_v[k][pl.ds(off, 16)] = accs[k]

    pltpu.async_copy(idx_slice(0), ib0, isem0)
    pltpu.async_copy(idx_slice(1), ib1, isem1)
    pltpu.make_async_copy(idx_slice(0), ib0, isem0).wait()
    fire_gathers(0, bufs[0])

    def outer(i, carry):
        for b in range(2):
            g = 2 * i + b
            nb = 1 - b
            drain_gathers(bufs[b])

            @pl.when(g + 2 < _NCH)
            def _fi():
                pltpu.async_copy(idx_slice(g + 2), bufs[b][0], bufs[b][2])

            @pl.when(g + 1 < _NCH)
            def _fg():
                pltpu.make_async_copy(idx_slice(0), bufs[nb][0],
                                      bufs[nb][2]).wait()
                fire_gathers(g + 1, bufs[nb])

            compute(g, bufs[b])
        return carry

    lax.fori_loop(0, _NCH // 2, outer, 0)
    base = pl.multiple_of(wid * _W, 8)
    for k in range(6):
        pltpu.sync_copy(res_v[k], res_out.at[pl.ds(k * _BS + base, _W)])
    pltpu.sync_copy(del_v, d_out.at[pl.ds(base, _W)])


def _tail_body(res_ref, delta_ref, edge_ref, hst_ref, mask_ref,
               sign_ref, loss_ref):
    r = res_ref[...]
    p_mu = -r[0:1]
    alpha = -r[1:1 + _H]
    m = jnp.max(alpha, axis=0, keepdims=True)
    es = jnp.exp(alpha - m)
    z_norm = jnp.sum(es, axis=0, keepdims=True)
    d_time = edge_ref[...] - hst_ref[...]
    wgt = (es / z_norm) * alpha * jnp.exp(-delta_ref[...] * d_time) * mask_ref[...]
    p_lambda = p_mu + jnp.sum(wgt, axis=0, keepdims=True)
    z = sign_ref[0] * p_lambda
    loss_ref[...] = jnp.maximum(-z, 0.0) + jnp.log(1.0 + jnp.exp(-jnp.abs(z)))


_sc_mesh = plsc.VectorSubcoreMesh(core_axis_name="c", subcore_axis_name="s")

_sc_dist = pl.kernel(
    _sc_body,
    out_type=(
        jax.ShapeDtypeStruct((6 * _BS,), jnp.float32),
        jax.ShapeDtypeStruct((_BS,), jnp.float32),
    ),
    mesh=_sc_mesh,
    compiler_params=pltpu.CompilerParams(needs_layout_passes=False),
    scratch_types=[
        pltpu.VMEM((_CR,), jnp.int32),
        pltpu.VMEM((_CR,), jnp.int32),
        pltpu.VMEM((_CR, _D), jnp.float32),
        pltpu.VMEM((_CR, _D), jnp.float32),
        pltpu.VMEM((_W,), jnp.float32),
        pltpu.VMEM((_W,), jnp.float32),
        pltpu.VMEM((_W,), jnp.float32),
        pltpu.VMEM((_W,), jnp.float32),
        pltpu.VMEM((_W,), jnp.float32),
        pltpu.VMEM((_W,), jnp.float32),
        pltpu.VMEM((_W,), jnp.float32),
        pltpu.SemaphoreType.DMA,
        pltpu.SemaphoreType.DMA,
        pltpu.SemaphoreType.DMA,
        pltpu.SemaphoreType.DMA,
    ],
)

_tail = pl.pallas_call(
    _tail_body,
    grid=(_TG,),
    in_specs=[
        pl.BlockSpec((6, _BLK), lambda i: (0, i)),
        pl.BlockSpec((1, _BLK), lambda i: (0, i)),
        pl.BlockSpec((1, _BLK), lambda i: (0, i)),
        pl.BlockSpec((_H, _BLK), lambda i: (0, i)),
        pl.BlockSpec((_H, _BLK), lambda i: (0, i)),
        pl.BlockSpec(memory_space=pltpu.SMEM),
    ],
    out_specs=pl.BlockSpec((1, _BLK), lambda i: (0, i)),
    out_shape=jax.ShapeDtypeStruct((1, _BS), jnp.float32),
)


def _make_idx(s32, t32, h32):
    s_r = s32.reshape(_NW * _NCH, 1, _C)
    t_r = t32.reshape(_NW * _NCH, 1, _C)
    h_r = h32.reshape(_NW * _NCH, _C, _H).transpose(0, 2, 1)
    return jnp.concatenate([s_r, t_r, h_r], axis=1).reshape(-1)


def kernel(sign, s, t, edge_times_batch, h_s, h_s_times, h_s_mask, emb_table,
           delta_table):
    def pad(x):
        return jnp.concatenate(
            [x, jnp.zeros((_BPAD - _B,) + x.shape[1:], x.dtype)], axis=0)

    def pad_idx(x):
        npad = _BPAD - x.shape[0]
        extra = (jnp.arange(npad * (x.size // x.shape[0]), dtype=jnp.int32)
                 % _N).reshape((npad,) + x.shape[1:])
        return jnp.concatenate([x.astype(jnp.int32), extra], axis=0)

    sp, tp, hp = pad_idx(s), pad_idx(t), pad_idx(h_s)
    hstp, maskp = pad(h_s_times), pad(h_s_mask)
    edgep = pad(edge_times_batch)
    dflat = delta_table.reshape(_N)

    losses = []
    for k in range(_S):
        lo = k * _BS
        idx_all = _make_idx(sp[lo:lo + _BS], tp[lo:lo + _BS], hp[lo:lo + _BS])
        res, delta_g = _sc_dist(emb_table, dflat, idx_all)
        losses.append(_tail(res.reshape(6, _BS), delta_g.reshape(1, _BS),
                            edgep[lo:lo + _BS].reshape(1, _BS),
                            hstp[lo:lo + _BS].T, maskp[lo:lo + _BS].T, sign))
    loss = jnp.concatenate(losses, axis=1)
    return loss[0, :_B]


# speedup vs baseline: 3.6810x; 3.3750x over previous
import jax
import jax.numpy as jnp
from jax import lax
from jax.experimental import pallas as pl
from jax.experimental.pallas import tpu as pltpu
from jax.experimental.pallas import tpu_sc as plsc

_N = 100000
_D = 128
_H = 5
_B = 100000
_R = _H + 2

_NC, _NS = 2, 16
_NW = _NC * _NS
_S = 4
_BPAD = 102400
_BS = _BPAD // _S
_W = _BS // _NW
_C = 40
_NCH = _W // _C
assert _NCH % 2 == 0 and _C % 8 == 0
_CR = _C * _R

_BLK = 640
_G = _BS // _BLK
assert _BLK % _C == 0 and _BS % _BLK == 0


def _gather_body(emb, dtab, idx_all, rows_out, d_out,
                 ib0, ib1, rows0, rows1, del_v,
                 isem0, isem1, gsem0, gsem1, wsem0, wsem1):
    wid = lax.axis_index("s") * _NC + lax.axis_index("c")
    ibase = pl.multiple_of(wid * _NCH * _CR, 8)
    dbase = pl.multiple_of(wid * _W, 8)
    bufs = ((ib0, rows0, isem0, gsem0, wsem0), (ib1, rows1, isem1, gsem1, wsem1))

    def idx_slice(g):
        return idx_all.at[pl.ds(ibase + g * _CR, _CR)]

    def fire_gathers(g, bs):
        ib, rows, _, gsem, _ = bs
        pltpu.async_copy(emb.at[ib.at[pl.ds(0, _C)]], rows.at[pl.ds(0, _C)], gsem)
        pltpu.async_copy(emb.at[ib.at[pl.ds(_C, _C)]], rows.at[pl.ds(_C, _C)], gsem)
        for j in range(_H):
            o = (2 + j) * _C
            pltpu.async_copy(emb.at[ib.at[pl.ds(o, _C)]], rows.at[pl.ds(o, _C)], gsem)
        pltpu.async_copy(dtab.at[ib.at[pl.ds(0, _C)]],
                         del_v.at[pl.ds(g * _C, _C)], gsem)

    def drain_gathers(bs):
        ib, rows, _, gsem, _ = bs
        for k in range(_R):
            pltpu.make_async_copy(emb.at[ib.at[pl.ds(0, _C)]],
                                  rows.at[pl.ds(k * _C, _C)], gsem).wait()
        pltpu.make_async_copy(dtab.at[ib.at[pl.ds(0, _C)]],
                              del_v.at[pl.ds(0, _C)], gsem).wait()

    pltpu.async_copy(idx_slice(0), ib0, isem0)
    pltpu.async_copy(idx_slice(1), ib1, isem1)

    def outer(i, carry):
        for b in range(2):
            g = 2 * i + b
            ib, rows, isem, gsem, wsem = bufs[b]

            @pl.when(g >= 2)
            def _dw():
                pltpu.make_async_copy(
                    rows, rows_out.at[pl.ds(ibase, _CR)], wsem).wait()

            pltpu.make_async_copy(idx_slice(0), ib, isem).wait()
            fire_gathers(g, bufs[b])
            drain_gathers(bufs[b])

            @pl.when(g + 2 < _NCH)
            def _fi():
                pltpu.async_copy(idx_slice(g + 2), ib, isem)

            pltpu.async_copy(rows, rows_out.at[pl.ds(ibase + g * _CR, _CR)], wsem)
        return carry

    lax.fori_loop(0, _NCH // 2, outer, 0)
    pltpu.make_async_copy(rows0, rows_out.at[pl.ds(ibase, _CR)], wsem0).wait()
    pltpu.make_async_copy(rows1, rows_out.at[pl.ds(ibase, _CR)], wsem1).wait()
    pltpu.sync_copy(del_v, d_out.at[pl.ds(dbase, _W)])


def _math_body(rows_ref, delta_ref, edge_ref, hst_ref, mask_ref,
               sign_ref, loss_ref):
    ps = []
    for c in range(_BLK // _C):
        b0 = c * _CR
        s_e = rows_ref[pl.ds(b0, _C)]
        t_e = rows_ref[pl.ds(b0 + _C, _C)]
        parts = [(s_e - t_e).astype(jnp.bfloat16)]
        for j in range(_H):
            he = rows_ref[pl.ds(b0 + (2 + j) * _C, _C)]
            parts.append((s_e - he).astype(jnp.bfloat16))
        ps.append(jnp.concatenate([q * q for q in parts], axis=1))
    p = jnp.concatenate(ps, axis=0)
    rows = lax.broadcasted_iota(jnp.int32, (6 * _D, 8), 0) // _D
    cols = lax.broadcasted_iota(jnp.int32, (6 * _D, 8), 1)
    w = (rows == cols).astype(jnp.bfloat16)
    r = lax.dot_general(p, w, (((1,), (0,)), ((), ())),
                        preferred_element_type=jnp.float32)
    rt = -r.T
    p_mu = rt[0:1]
    alpha = rt[1:1 + _H]
    m = jnp.max(alpha, axis=0, keepdims=True)
    es = jnp.exp(alpha - m)
    z_norm = jnp.sum(es, axis=0, keepdims=True)
    d_time = edge_ref[...] - hst_ref[...]
    wgt = (es / z_norm) * alpha * jnp.exp(-delta_ref[...] * d_time) * mask_ref[...]
    p_lambda = p_mu + jnp.sum(wgt, axis=0, keepdims=True)
    z = sign_ref[0] * p_lambda
    loss_ref[...] = jnp.maximum(-z, 0.0) + jnp.log(1.0 + jnp.exp(-jnp.abs(z)))


_sc_mesh = plsc.VectorSubcoreMesh(core_axis_name="c", subcore_axis_name="s")

_gather = pl.kernel(
    _gather_body,
    out_type=(
        jax.ShapeDtypeStruct((_BS * _R, _D), jnp.float32),
        jax.ShapeDtypeStruct((_BS,), jnp.float32),
    ),
    mesh=_sc_mesh,
    scratch_types=[
        pltpu.VMEM((_CR,), jnp.int32),
        pltpu.VMEM((_CR,), jnp.int32),
        pltpu.VMEM((_CR, _D), jnp.float32),
        pltpu.VMEM((_CR, _D), jnp.float32),
        pltpu.VMEM((_W,), jnp.float32),
        pltpu.SemaphoreType.DMA,
        pltpu.SemaphoreType.DMA,
        pltpu.SemaphoreType.DMA,
        pltpu.SemaphoreType.DMA,
        pltpu.SemaphoreType.DMA,
        pltpu.SemaphoreType.DMA,
    ],
)

_math = pl.pallas_call(
    _math_body,
    grid=(_G,),
    in_specs=[
        pl.BlockSpec((_BLK * _R, _D), lambda i: (i, 0)),
        pl.BlockSpec((1, _BLK), lambda i: (0, i)),
        pl.BlockSpec((1, _BLK), lambda i: (0, i)),
        pl.BlockSpec((_H, _BLK), lambda i: (0, i)),
        pl.BlockSpec((_H, _BLK), lambda i: (0, i)),
        pl.BlockSpec(memory_space=pltpu.SMEM),
    ],
    out_specs=pl.BlockSpec((1, _BLK), lambda i: (0, i)),
    out_shape=jax.ShapeDtypeStruct((1, _BS), jnp.float32),
)


def _make_idx(s32, t32, h32):
    s_r = s32.reshape(_NW * _NCH, 1, _C)
    t_r = t32.reshape(_NW * _NCH, 1, _C)
    h_r = h32.reshape(_NW * _NCH, _C, _H).transpose(0, 2, 1)
    return jnp.concatenate([s_r, t_r, h_r], axis=1).reshape(-1)


def kernel(sign, s, t, edge_times_batch, h_s, h_s_times, h_s_mask, emb_table,
           delta_table):
    def pad(x):
        return jnp.concatenate(
            [x, jnp.zeros((_BPAD - _B,) + x.shape[1:], x.dtype)], axis=0)

    def pad_idx(x):
        npad = _BPAD - x.shape[0]
        extra = (jnp.arange(npad * (x.size // x.shape[0]), dtype=jnp.int32)
                 % _N).reshape((npad,) + x.shape[1:])
        return jnp.concatenate([x.astype(jnp.int32), extra], axis=0)

    sp, tp, hp = pad_idx(s), pad_idx(t), pad_idx(h_s)
    hstp, maskp = pad(h_s_times), pad(h_s_mask)
    edgep = pad(edge_times_batch)
    dflat = delta_table.reshape(_N)

    losses = []
    for k in range(_S):
        lo = k * _BS
        idx_all = _make_idx(sp[lo:lo + _BS], tp[lo:lo + _BS], hp[lo:lo + _BS])
        rows_all, delta_g = _gather(emb_table, dflat, idx_all)
        losses.append(_math(rows_all, delta_g.reshape(1, _BS),
                            edgep[lo:lo + _BS].reshape(1, _BS),
                            hstp[lo:lo + _BS].T, maskp[lo:lo + _BS].T, sign))
    loss = jnp.concatenate(losses, axis=1)
    return loss[0, :_B]
